# baseline (speedup 1.0000x reference)
import jax
import jax.numpy as jnp
from jax.experimental import pallas as pl
from jax.experimental.pallas import tpu as pltpu

B, S, D, F, E, K = 32, 4, 1024, 512, 64, 2
T = B * S
FH = F // 2


def _body(x_ref, wg_ref, sg_ref, su_ref, sd_ref, wgate_ref, wup_ref,
          wdown_ref, out_ref, idx_scr, wt_scr):
    e = pl.program_id(0)
    j = pl.program_id(1)

    @pl.when((e == 0) & (j == 0))
    def _router_and_shared():
        x = x_ref[...]
        logits = jax.lax.dot_general(x, wg_ref[...],
                                     (((1,), (1,)), ((), ())),
                                     preferred_element_type=jnp.float32)
        m = jnp.max(logits, axis=1, keepdims=True)
        p = jnp.exp(logits - m)
        p = p / jnp.sum(p, axis=1, keepdims=True)
        eidx = jax.lax.broadcasted_iota(jnp.int32, (T, E), 1)
        v1 = jnp.max(p, axis=1, keepdims=True)
        i1 = jnp.min(jnp.where(p == v1, eidx, E), axis=1, keepdims=True)
        p2 = jnp.where(eidx == i1, -1.0, p)
        v2 = jnp.max(p2, axis=1, keepdims=True)
        i2 = jnp.min(jnp.where(p2 == v2, eidx, E), axis=1, keepdims=True)
        s = v1 + v2
        idx_scr[:, 0:1] = i1
        idx_scr[:, 1:2] = i2
        wt_scr[:, 0:1] = v1 / s
        wt_scr[:, 1:2] = v2 / s
        xb = x.astype(jnp.bfloat16)
        g = jax.lax.dot_general(xb, sg_ref[...].astype(jnp.bfloat16),
                                (((1,), (1,)), ((), ())),
                                preferred_element_type=jnp.float32)
        u = jax.lax.dot_general(xb, su_ref[...].astype(jnp.bfloat16),
                                (((1,), (1,)), ((), ())),
                                preferred_element_type=jnp.float32)
        a = (jax.nn.silu(g) * u).astype(jnp.bfloat16)
        out_ref[...] = jax.lax.dot_general(
            a, sd_ref[...].astype(jnp.bfloat16), (((1,), (1,)), ((), ())),
            preferred_element_type=jnp.float32)

    x = x_ref[...].astype(jnp.bfloat16)
    h = jax.lax.dot_general(x, wgate_ref[0].astype(jnp.bfloat16),
                            (((1,), (1,)), ((), ())),
                            preferred_element_type=jnp.float32)
    u = jax.lax.dot_general(x, wup_ref[0].astype(jnp.bfloat16),
                            (((1,), (1,)), ((), ())),
                            preferred_element_type=jnp.float32)
    a = (jax.nn.silu(h) * u).astype(jnp.bfloat16)
    y = jax.lax.dot_general(a, wdown_ref[0].astype(jnp.bfloat16),
                            (((1,), (1,)), ((), ())),
                            preferred_element_type=jnp.float32)
    c = (jnp.where(idx_scr[:, 0:1] == e, wt_scr[:, 0:1], 0.0)
         + jnp.where(idx_scr[:, 1:2] == e, wt_scr[:, 1:2], 0.0))
    out_ref[...] += y * c


def kernel(hidden_states, wg, w_gate, w_up, w_down, shared_gate, shared_up,
           shared_down):
    x = hidden_states.reshape(T, D)
    out = pl.pallas_call(
        _body,
        grid=(E, 2),
        in_specs=[
            pl.BlockSpec((T, D), lambda e, j: (0, 0)),
            pl.BlockSpec((E, D), lambda e, j: (0, 0)),
            pl.BlockSpec((F, D), lambda e, j: (0, 0)),
            pl.BlockSpec((F, D), lambda e, j: (0, 0)),
            pl.BlockSpec((D, F), lambda e, j: (0, 0)),
            pl.BlockSpec((1, FH, D), lambda e, j: (e, j, 0)),
            pl.BlockSpec((1, FH, D), lambda e, j: (e, j, 0)),
            pl.BlockSpec((1, D, FH), lambda e, j: (e, 0, j)),
        ],
        out_specs=pl.BlockSpec((T, D), lambda e, j: (0, 0)),
        out_shape=jax.ShapeDtypeStruct((T, D), jnp.float32),
        scratch_shapes=[
            pltpu.VMEM((T, 128), jnp.int32),
            pltpu.VMEM((T, 128), jnp.float32),
        ],
    )(x, wg, shared_gate, shared_up, shared_down, w_gate, w_up, w_down)
    return out.reshape(B, S, D)


# speedup vs baseline: 1.3994x; 1.3994x over previous
import jax
import jax.numpy as jnp
from jax.experimental import pallas as pl
from jax.experimental.pallas import tpu as pltpu

B, S, D, F, E, K = 32, 4, 1024, 512, 64, 2
T = B * S
EPB = 2


def _body(x_ref, wg_ref, sg_ref, su_ref, sd_ref, wgate_ref, wup_ref,
          wdown_ref, out_ref, idx_scr, wt_scr):
    e = pl.program_id(0)

    @pl.when(e == 0)
    def _router_and_shared():
        x = x_ref[...]
        logits = jax.lax.dot_general(x, wg_ref[...],
                                     (((1,), (1,)), ((), ())),
                                     preferred_element_type=jnp.float32)
        m = jnp.max(logits, axis=1, keepdims=True)
        p = jnp.exp(logits - m)
        p = p / jnp.sum(p, axis=1, keepdims=True)
        eidx = jax.lax.broadcasted_iota(jnp.int32, (T, E), 1)
        v1 = jnp.max(p, axis=1, keepdims=True)
        i1 = jnp.min(jnp.where(p == v1, eidx, E), axis=1, keepdims=True)
        p2 = jnp.where(eidx == i1, -1.0, p)
        v2 = jnp.max(p2, axis=1, keepdims=True)
        i2 = jnp.min(jnp.where(p2 == v2, eidx, E), axis=1, keepdims=True)
        s = v1 + v2
        idx_scr[:, 0:1] = i1
        idx_scr[:, 1:2] = i2
        wt_scr[:, 0:1] = v1 / s
        wt_scr[:, 1:2] = v2 / s
        xb = x.astype(jnp.bfloat16)
        g = jax.lax.dot_general(xb, sg_ref[...].astype(jnp.bfloat16),
                                (((1,), (1,)), ((), ())),
                                preferred_element_type=jnp.float32)
        u = jax.lax.dot_general(xb, su_ref[...].astype(jnp.bfloat16),
                                (((1,), (1,)), ((), ())),
                                preferred_element_type=jnp.float32)
        a = (jax.nn.silu(g) * u).astype(jnp.bfloat16)
        out_ref[...] = jax.lax.dot_general(
            a, sd_ref[...].astype(jnp.bfloat16), (((1,), (1,)), ((), ())),
            preferred_element_type=jnp.float32)

    x = x_ref[...].astype(jnp.bfloat16)
    acc = jnp.zeros((T, D), jnp.float32)
    for k in range(EPB):
        eid = e * EPB + k
        h = jax.lax.dot_general(x, wgate_ref[k].astype(jnp.bfloat16),
                                (((1,), (1,)), ((), ())),
                                preferred_element_type=jnp.float32)
        u = jax.lax.dot_general(x, wup_ref[k].astype(jnp.bfloat16),
                                (((1,), (1,)), ((), ())),
                                preferred_element_type=jnp.float32)
        a = (jax.nn.silu(h) * u).astype(jnp.bfloat16)
        y = jax.lax.dot_general(a, wdown_ref[k].astype(jnp.bfloat16),
                                (((1,), (1,)), ((), ())),
                                preferred_element_type=jnp.float32)
        c = (jnp.where(idx_scr[:, 0:1] == eid, wt_scr[:, 0:1], 0.0)
             + jnp.where(idx_scr[:, 1:2] == eid, wt_scr[:, 1:2], 0.0))
        acc = acc + y * c
    out_ref[...] += acc


def kernel(hidden_states, wg, w_gate, w_up, w_down, shared_gate, shared_up,
           shared_down):
    x = hidden_states.reshape(T, D)
    out = pl.pallas_call(
        _body,
        grid=(E // EPB,),
        in_specs=[
            pl.BlockSpec((T, D), lambda e: (0, 0)),
            pl.BlockSpec((E, D), lambda e: (0, 0)),
            pl.BlockSpec((F, D), lambda e: (0, 0)),
            pl.BlockSpec((F, D), lambda e: (0, 0)),
            pl.BlockSpec((D, F), lambda e: (0, 0)),
            pl.BlockSpec((EPB, F, D), lambda e: (e, 0, 0)),
            pl.BlockSpec((EPB, F, D), lambda e: (e, 0, 0)),
            pl.BlockSpec((EPB, D, F), lambda e: (e, 0, 0)),
        ],
        out_specs=pl.BlockSpec((T, D), lambda e: (0, 0)),
        out_shape=jax.ShapeDtypeStruct((T, D), jnp.float32),
        scratch_shapes=[
            pltpu.VMEM((T, 128), jnp.int32),
            pltpu.VMEM((T, 128), jnp.float32),
        ],
    )(x, wg, shared_gate, shared_up, shared_down, w_gate, w_up, w_down)
    return out.reshape(B, S, D)
